# baseline (speedup 1.0000x reference)
import functools
import math

import jax
import jax.numpy as jnp
from jax.experimental import pallas as pl
from jax.experimental.pallas import tpu as pltpu

B, N, C = 1, 2048, 768
H = 12
HD = C // H
BLKQ = 64
BLKK = 64
NQ = N // BLKQ
NK = N // BLKK
KSEL = max(1, int(0.25 * NK))
SCALE = 1.0 / math.sqrt(HD)

ROW_TILE = 256
N_ROW_TILES = N // ROW_TILE
POOL_PER_TILE = ROW_TILE // BLKQ
G = H // 2
LN_ROWS = N // 2


def _row_softmax(a):
    m = jnp.max(a, axis=1, keepdims=True)
    e = jnp.exp(a - m)
    return e / jnp.sum(e, axis=1, keepdims=True)


def _qkv_topk_kernel(x_ref, w_ref, b_ref, qkv_ref, idx_ref, pacc_ref):
    r = pl.program_id(0)
    t = jax.lax.dot_general(x_ref[:, :], w_ref[:, :], (((1,), (1,)), ((), ())),
                            preferred_element_type=jnp.float32) + b_ref[:, :]
    qkv_ref[:, :] = t
    qk = t[:, :2 * C]
    for b in range(POOL_PER_TILE):
        row = (jnp.sum(qk[b * BLKQ:(b + 1) * BLKQ, :], axis=0) * (1.0 / BLKQ))
        pacc_ref[pl.ds(r * POOL_PER_TILE + b, 1), :] = row[None, :]

    @pl.when(r == N_ROW_TILES - 1)
    def _():
        col = jax.lax.broadcasted_iota(jnp.int32, (NQ, NK), 1)
        for h in range(H):
            qp = pacc_ref[:, h * HD:(h + 1) * HD]
            kp = pacc_ref[:, C + h * HD:C + (h + 1) * HD]
            s = jax.lax.dot_general(qp, kp, (((1,), (1,)), ((), ())),
                                    preferred_element_type=jnp.float32)
            js = []
            for t_ in range(KSEL):
                m = jnp.max(s, axis=1, keepdims=True)
                j = jnp.min(jnp.where(s == m, col, NK), axis=1)
                js.append(j)
                s = jnp.where(col == j[:, None], -jnp.inf, s)
            idx_ref[h, :, :] = jnp.stack(js, axis=1)


def _attn_ln_proj_kernel(idx_ref, q_ref, k_ref, v_ref, g_ref, bt_ref, wp_ref,
                         bp_ref, y_ref, x2_ref, pq_ref, vaug_ref, kvb_ref):
    g = pl.program_id(0)

    @pl.when(g < G)
    def _attn():
        kvtot_augs = []
        for hh in range(2):
            lo, hi = hh * HD, (hh + 1) * HD
            k = k_ref[:, lo:hi]
            v = v_ref[:, lo:hi]
            phi = _row_softmax(k)
            pq_ref[:, lo:hi] = _row_softmax(q_ref[:, lo:hi])
            vaug_ref[hh, :, :] = jnp.concatenate(
                [v, jnp.ones((N, HD), jnp.float32)], axis=1)
            kv_tot = jax.lax.dot_general(phi, v, (((0,), (0,)), ((), ())),
                                         preferred_element_type=jnp.float32)
            s_tot = jnp.sum(phi, axis=0)
            kvtot_augs.append(jnp.concatenate(
                [kv_tot, jnp.broadcast_to(s_tot[:, None], (HD, HD))], axis=1))
            for j in range(NK):
                phi_j = phi[j * BLKK:(j + 1) * BLKK, :]
                v_j = v[j * BLKK:(j + 1) * BLKK, :]
                kv_j = jax.lax.dot_general(phi_j, v_j, (((0,), (0,)), ((), ())),
                                           preferred_element_type=jnp.float32)
                s_j = jnp.sum(phi_j, axis=0)
                kvb_ref[hh, pl.ds(j * BLKK, BLKK), :] = jnp.concatenate(
                    [kv_j, jnp.broadcast_to(s_j[:, None], (HD, HD))], axis=1)

        def body(i, _):
            for hh in range(2):
                lo, hi = hh * HD, (hh + 1) * HD
                q_i = q_ref[pl.ds(i * BLKQ, BLKQ), lo:hi]
                idxs = [idx_ref[2 * g + hh, i, t] for t in range(KSEL)]
                k_sel = jnp.concatenate(
                    [k_ref[pl.ds(j * BLKK, BLKK), lo:hi] for j in idxs], axis=0)
                vaug_sel = jnp.concatenate(
                    [vaug_ref[hh, pl.ds(j * BLKK, BLKK), :] for j in idxs], axis=0)
                s = jax.lax.dot_general(q_i, k_sel, (((1,), (1,)), ((), ())),
                                        preferred_element_type=jnp.float32) * SCALE
                e = jnp.exp(s)
                out_aug = jax.lax.dot_general(e, vaug_sel, (((1,), (0,)), ((), ())),
                                              preferred_element_type=jnp.float32)
                kv_aug = kvb_ref[hh, pl.ds(idxs[0] * BLKK, BLKK), :]
                for j in idxs[1:]:
                    kv_aug = kv_aug + kvb_ref[hh, pl.ds(j * BLKK, BLKK), :]
                pq_i = pq_ref[pl.ds(i * BLKQ, BLKQ), lo:hi]
                out_l_aug = jax.lax.dot_general(pq_i, kvtot_augs[hh] - kv_aug,
                                                (((1,), (0,)), ((), ())),
                                                preferred_element_type=jnp.float32)
                tot = out_aug + out_l_aug
                x2_ref[g, pl.ds(i * BLKQ, BLKQ), lo:hi] = (
                    tot[:, :HD] / tot[:, HD:HD + 1])
            return 0

        jax.lax.fori_loop(0, NQ, body, 0)

    @pl.when(g >= G)
    def _ln_proj():
        half = g - G
        xb = jnp.concatenate(
            [x2_ref[gg, pl.ds(half * LN_ROWS, LN_ROWS), :] for gg in range(G)],
            axis=1)
        mu = jnp.mean(xb, axis=1, keepdims=True)
        d = xb - mu
        var = jnp.mean(d * d, axis=1, keepdims=True)
        xn = d / jnp.sqrt(var + 1e-5) * g_ref[:, :] + bt_ref[:, :]
        y_ref[:, :] = jax.lax.dot_general(
            xn, wp_ref[:, :], (((1,), (1,)), ((), ())),
            preferred_element_type=jnp.float32) + bp_ref[:, :]


@functools.partial(jax.jit, static_argnames=("interpret",))
def _run(x, W_qkv, q_bias, v_bias, gamma, beta, W_proj, b_proj, interpret=False):
    x2d = x.reshape(N, C)
    qkv_bias = jnp.concatenate(
        [q_bias, jnp.zeros_like(q_bias), v_bias]).reshape(1, 3 * C)

    qkv, idx = pl.pallas_call(
        _qkv_topk_kernel,
        grid=(N_ROW_TILES,),
        in_specs=[
            pl.BlockSpec((ROW_TILE, C), lambda r: (r, 0)),
            pl.BlockSpec((3 * C, C), lambda r: (0, 0)),
            pl.BlockSpec((1, 3 * C), lambda r: (0, 0)),
        ],
        out_specs=[
            pl.BlockSpec((ROW_TILE, 3 * C), lambda r: (r, 0)),
            pl.BlockSpec((H, NQ, KSEL), lambda r: (0, 0, 0)),
        ],
        out_shape=[
            jax.ShapeDtypeStruct((N, 3 * C), jnp.float32),
            jax.ShapeDtypeStruct((H, NQ, KSEL), jnp.int32),
        ],
        scratch_shapes=[pltpu.VMEM((NQ, 2 * C), jnp.float32)],
        interpret=interpret,
    )(x2d, W_qkv, qkv_bias)

    y = pl.pallas_call(
        _attn_ln_proj_kernel,
        grid_spec=pltpu.PrefetchScalarGridSpec(
            num_scalar_prefetch=1,
            grid=(G + 2,),
            in_specs=[
                pl.BlockSpec((N, 2 * HD), lambda g, s: (0, jnp.minimum(g, G - 1))),
                pl.BlockSpec((N, 2 * HD), lambda g, s: (0, G + jnp.minimum(g, G - 1))),
                pl.BlockSpec((N, 2 * HD), lambda g, s: (0, 2 * G + jnp.minimum(g, G - 1))),
                pl.BlockSpec((1, C), lambda g, s: (0, 0)),
                pl.BlockSpec((1, C), lambda g, s: (0, 0)),
                pl.BlockSpec((C, C), lambda g, s: (0, 0)),
                pl.BlockSpec((1, C), lambda g, s: (0, 0)),
            ],
            out_specs=pl.BlockSpec((LN_ROWS, C),
                                   lambda g, s: (jnp.maximum(g - G, 0), 0)),
            scratch_shapes=[
                pltpu.VMEM((G, N, 2 * HD), jnp.float32),
                pltpu.VMEM((N, 2 * HD), jnp.float32),
                pltpu.VMEM((2, N, 2 * HD), jnp.float32),
                pltpu.VMEM((2, N, 2 * HD), jnp.float32),
            ],
        ),
        out_shape=jax.ShapeDtypeStruct((N, C), jnp.float32),
        interpret=interpret,
    )(idx, qkv, qkv, qkv, gamma.reshape(1, C), beta.reshape(1, C),
      W_proj, b_proj.reshape(1, C))

    return y.reshape(B, N, C)


def kernel(x, W_qkv, q_bias, v_bias, gamma, beta, W_proj, b_proj):
    return _run(x, W_qkv, q_bias, v_bias, gamma, beta, W_proj, b_proj)


# speedup vs baseline: 1.1639x; 1.1639x over previous
import functools
import math

import jax
import jax.numpy as jnp
from jax.experimental import pallas as pl
from jax.experimental.pallas import tpu as pltpu

B, N, C = 1, 2048, 768
H = 12
HD = C // H
BLKQ = 64
BLKK = 64
NQ = N // BLKQ
NK = N // BLKK
KSEL = max(1, int(0.25 * NK))
SCALE = 1.0 / math.sqrt(HD)

ROW_TILE = 256
N_ROW_TILES = N // ROW_TILE
POOL_PER_TILE = ROW_TILE // BLKQ
G = H // 2
LN_ROWS = N // 2


def _row_softmax(a):
    m = jnp.max(a, axis=1, keepdims=True)
    e = jnp.exp(a - m)
    return e / jnp.sum(e, axis=1, keepdims=True)


def _qkv_topk_kernel(x_ref, w_ref, b_ref, qkv_ref, idx_ref, pacc_ref):
    r = pl.program_id(0)
    t = jax.lax.dot_general(x_ref[:, :], w_ref[:, :], (((1,), (1,)), ((), ())),
                            preferred_element_type=jnp.float32) + b_ref[:, :]
    qkv_ref[:, :] = t
    qk = t[:, :2 * C]
    for b in range(POOL_PER_TILE):
        row = (jnp.sum(qk[b * BLKQ:(b + 1) * BLKQ, :], axis=0) * (1.0 / BLKQ))
        pacc_ref[pl.ds(r * POOL_PER_TILE + b, 1), :] = row[None, :]

    @pl.when(r == N_ROW_TILES - 1)
    def _():
        col = jax.lax.broadcasted_iota(jnp.int32, (NQ, NK), 1)
        for h in range(H):
            qp = pacc_ref[:, h * HD:(h + 1) * HD]
            kp = pacc_ref[:, C + h * HD:C + (h + 1) * HD]
            s = jax.lax.dot_general(qp, kp, (((1,), (1,)), ((), ())),
                                    preferred_element_type=jnp.float32)
            js = []
            for t_ in range(KSEL):
                m = jnp.max(s, axis=1, keepdims=True)
                j = jnp.min(jnp.where(s == m, col, NK), axis=1)
                js.append(j)
                s = jnp.where(col == j[:, None], -jnp.inf, s)
            idx_ref[h, :, :] = jnp.stack(js, axis=1)


def _attn_ln_proj_kernel(idx_ref, q_ref, k_ref, v_ref, g_ref, bt_ref, wp_ref,
                         bp_ref, y_ref, x2_ref, pq_ref, vaug_ref, kvb_ref):
    g = pl.program_id(0)

    @pl.when(g < G)
    def _attn():
        kvtot_augs = []
        for hh in range(2):
            lo, hi = hh * HD, (hh + 1) * HD
            k = k_ref[:, lo:hi]
            v = v_ref[:, lo:hi]
            phi = _row_softmax(k)
            pq_ref[:, lo:hi] = _row_softmax(q_ref[:, lo:hi])
            vaug_ref[hh, :, :] = jnp.concatenate(
                [v, jnp.ones((N, HD), jnp.float32)], axis=1)
            kv_tot = jax.lax.dot_general(phi, v, (((0,), (0,)), ((), ())),
                                         preferred_element_type=jnp.float32)
            s_tot = jnp.sum(phi, axis=0)
            kvtot_augs.append(jnp.concatenate(
                [kv_tot, jnp.broadcast_to(s_tot[:, None], (HD, HD))], axis=1))
            for j in range(NK):
                phi_j = phi[j * BLKK:(j + 1) * BLKK, :]
                v_j = v[j * BLKK:(j + 1) * BLKK, :]
                kv_j = jax.lax.dot_general(phi_j, v_j, (((0,), (0,)), ((), ())),
                                           preferred_element_type=jnp.float32)
                s_j = jnp.sum(phi_j, axis=0)
                kvb_ref[hh, pl.ds(j * BLKK, BLKK), :] = jnp.concatenate(
                    [kv_j, jnp.broadcast_to(s_j[:, None], (HD, HD))], axis=1)

        def body(i2, _):
            for half in range(2):
              i = i2 * 2 + half
              for hh in range(2):
                lo, hi = hh * HD, (hh + 1) * HD
                q_i = q_ref[pl.ds(i * BLKQ, BLKQ), lo:hi]
                idxs = [idx_ref[2 * g + hh, i, t] for t in range(KSEL)]
                k_sel = jnp.concatenate(
                    [k_ref[pl.ds(j * BLKK, BLKK), lo:hi] for j in idxs], axis=0)
                vaug_sel = jnp.concatenate(
                    [vaug_ref[hh, pl.ds(j * BLKK, BLKK), :] for j in idxs], axis=0)
                s = jax.lax.dot_general(q_i, k_sel, (((1,), (1,)), ((), ())),
                                        preferred_element_type=jnp.float32) * SCALE
                e = jnp.exp(s)
                out_aug = jax.lax.dot_general(e, vaug_sel, (((1,), (0,)), ((), ())),
                                              preferred_element_type=jnp.float32)
                kv_aug = kvb_ref[hh, pl.ds(idxs[0] * BLKK, BLKK), :]
                for j in idxs[1:]:
                    kv_aug = kv_aug + kvb_ref[hh, pl.ds(j * BLKK, BLKK), :]
                pq_i = pq_ref[pl.ds(i * BLKQ, BLKQ), lo:hi]
                out_l_aug = jax.lax.dot_general(pq_i, kvtot_augs[hh] - kv_aug,
                                                (((1,), (0,)), ((), ())),
                                                preferred_element_type=jnp.float32)
                tot = out_aug + out_l_aug
                x2_ref[g, pl.ds(i * BLKQ, BLKQ), lo:hi] = (
                    tot[:, :HD] / tot[:, HD:HD + 1])
            return 0

        jax.lax.fori_loop(0, NQ // 2, body, 0)

    @pl.when(g >= G)
    def _ln_proj():
        half = g - G
        xb = jnp.concatenate(
            [x2_ref[gg, pl.ds(half * LN_ROWS, LN_ROWS), :] for gg in range(G)],
            axis=1)
        mu = jnp.mean(xb, axis=1, keepdims=True)
        d = xb - mu
        var = jnp.mean(d * d, axis=1, keepdims=True)
        xn = d / jnp.sqrt(var + 1e-5) * g_ref[:, :] + bt_ref[:, :]
        y_ref[:, :] = jax.lax.dot_general(
            xn, wp_ref[:, :], (((1,), (1,)), ((), ())),
            preferred_element_type=jnp.float32) + bp_ref[:, :]


@functools.partial(jax.jit, static_argnames=("interpret",))
def _run(x, W_qkv, q_bias, v_bias, gamma, beta, W_proj, b_proj, interpret=False):
    x2d = x.reshape(N, C)
    qkv_bias = jnp.concatenate(
        [q_bias, jnp.zeros_like(q_bias), v_bias]).reshape(1, 3 * C)

    qkv, idx = pl.pallas_call(
        _qkv_topk_kernel,
        grid=(N_ROW_TILES,),
        in_specs=[
            pl.BlockSpec((ROW_TILE, C), lambda r: (r, 0)),
            pl.BlockSpec((3 * C, C), lambda r: (0, 0)),
            pl.BlockSpec((1, 3 * C), lambda r: (0, 0)),
        ],
        out_specs=[
            pl.BlockSpec((ROW_TILE, 3 * C), lambda r: (r, 0)),
            pl.BlockSpec((H, NQ, KSEL), lambda r: (0, 0, 0)),
        ],
        out_shape=[
            jax.ShapeDtypeStruct((N, 3 * C), jnp.float32),
            jax.ShapeDtypeStruct((H, NQ, KSEL), jnp.int32),
        ],
        scratch_shapes=[pltpu.VMEM((NQ, 2 * C), jnp.float32)],
        interpret=interpret,
    )(x2d, W_qkv, qkv_bias)

    y = pl.pallas_call(
        _attn_ln_proj_kernel,
        grid_spec=pltpu.PrefetchScalarGridSpec(
            num_scalar_prefetch=1,
            grid=(G + 2,),
            in_specs=[
                pl.BlockSpec((N, 2 * HD), lambda g, s: (0, jnp.minimum(g, G - 1))),
                pl.BlockSpec((N, 2 * HD), lambda g, s: (0, G + jnp.minimum(g, G - 1))),
                pl.BlockSpec((N, 2 * HD), lambda g, s: (0, 2 * G + jnp.minimum(g, G - 1))),
                pl.BlockSpec((1, C), lambda g, s: (0, 0)),
                pl.BlockSpec((1, C), lambda g, s: (0, 0)),
                pl.BlockSpec((C, C), lambda g, s: (0, 0)),
                pl.BlockSpec((1, C), lambda g, s: (0, 0)),
            ],
            out_specs=pl.BlockSpec((LN_ROWS, C),
                                   lambda g, s: (jnp.maximum(g - G, 0), 0)),
            scratch_shapes=[
                pltpu.VMEM((G, N, 2 * HD), jnp.float32),
                pltpu.VMEM((N, 2 * HD), jnp.float32),
                pltpu.VMEM((2, N, 2 * HD), jnp.float32),
                pltpu.VMEM((2, N, 2 * HD), jnp.float32),
            ],
        ),
        out_shape=jax.ShapeDtypeStruct((N, C), jnp.float32),
        interpret=interpret,
    )(idx, qkv, qkv, qkv, gamma.reshape(1, C), beta.reshape(1, C),
      W_proj, b_proj.reshape(1, C))

    return y.reshape(B, N, C)


def kernel(x, W_qkv, q_bias, v_bias, gamma, beta, W_proj, b_proj):
    return _run(x, W_qkv, q_bias, v_bias, gamma, beta, W_proj, b_proj)


# speedup vs baseline: 1.2608x; 1.0832x over previous
import functools
import math

import jax
import jax.numpy as jnp
from jax.experimental import pallas as pl
from jax.experimental.pallas import tpu as pltpu

B, N, C = 1, 2048, 768
H = 12
HD = C // H
BLKQ = 64
BLKK = 64
NQ = N // BLKQ
NK = N // BLKK
KSEL = max(1, int(0.25 * NK))
SCALE = 1.0 / math.sqrt(HD)

ROW_TILE = 256
N_ROW_TILES = N // ROW_TILE
POOL_PER_TILE = ROW_TILE // BLKQ
G = H // 2
LN_ROWS = N // 2


def _row_softmax(a):
    m = jnp.max(a, axis=1, keepdims=True)
    e = jnp.exp(a - m)
    return e / jnp.sum(e, axis=1, keepdims=True)


def _qkv_topk_kernel(x_ref, w_ref, b_ref, qkv_ref, idx_ref, pacc_ref):
    r = pl.program_id(0)
    t = jax.lax.dot_general(x_ref[:, :], w_ref[:, :], (((1,), (1,)), ((), ())),
                            preferred_element_type=jnp.float32) + b_ref[:, :]
    qkv_ref[:, :] = t
    qk = t[:, :2 * C]
    for b in range(POOL_PER_TILE):
        row = (jnp.sum(qk[b * BLKQ:(b + 1) * BLKQ, :], axis=0) * (1.0 / BLKQ))
        pacc_ref[pl.ds(r * POOL_PER_TILE + b, 1), :] = row[None, :]

    @pl.when(r == N_ROW_TILES - 1)
    def _():
        col = jax.lax.broadcasted_iota(jnp.int32, (NQ, NK), 1)
        for h in range(H):
            qp = pacc_ref[:, h * HD:(h + 1) * HD]
            kp = pacc_ref[:, C + h * HD:C + (h + 1) * HD]
            s = jax.lax.dot_general(qp, kp, (((1,), (1,)), ((), ())),
                                    preferred_element_type=jnp.float32)
            js = []
            for t_ in range(KSEL):
                m = jnp.max(s, axis=1, keepdims=True)
                j = jnp.min(jnp.where(s == m, col, NK), axis=1)
                js.append(j)
                s = jnp.where(col == j[:, None], -jnp.inf, s)
            idx_ref[h, :, :] = jnp.stack(js, axis=1)


def _attn_ln_proj_kernel(idx_ref, q_ref, k_ref, v_ref, g_ref, bt_ref, wp_ref,
                         bp_ref, y_ref, x2_ref, pq_ref, vaug_ref, kvb_ref):
    g = pl.program_id(0)

    @pl.when(g < G)
    def _attn():
        kvtot_augs = []
        for hh in range(2):
            lo, hi = hh * HD, (hh + 1) * HD
            k = k_ref[:, lo:hi]
            v = v_ref[:, lo:hi]
            phi = _row_softmax(k)
            pq_ref[:, lo:hi] = _row_softmax(q_ref[:, lo:hi])
            vaug_ref[hh, :, :] = jnp.concatenate(
                [v, jnp.ones((N, HD), jnp.float32)], axis=1)
            kv_tot = jax.lax.dot_general(phi, v, (((0,), (0,)), ((), ())),
                                         preferred_element_type=jnp.float32)
            s_tot = jnp.sum(phi, axis=0)
            kvtot_augs.append(jnp.concatenate(
                [kv_tot, jnp.broadcast_to(s_tot[:, None], (HD, HD))], axis=1))
            for j in range(NK):
                phi_j = phi[j * BLKK:(j + 1) * BLKK, :]
                v_j = v[j * BLKK:(j + 1) * BLKK, :]
                kv_j = jax.lax.dot_general(phi_j, v_j, (((0,), (0,)), ((), ())),
                                           preferred_element_type=jnp.float32)
                s_j = jnp.sum(phi_j, axis=0)
                kvb_ref[hh, pl.ds(j * BLKK, BLKK), :] = jnp.concatenate(
                    [kv_j, jnp.broadcast_to(s_j[:, None], (HD, HD))], axis=1)

        def body(i2, _):
            for half in range(4):
              i = i2 * 4 + half
              for hh in range(2):
                lo, hi = hh * HD, (hh + 1) * HD
                q_i = q_ref[pl.ds(i * BLKQ, BLKQ), lo:hi]
                idxs = [idx_ref[2 * g + hh, i, t] for t in range(KSEL)]
                k_sel = jnp.concatenate(
                    [k_ref[pl.ds(j * BLKK, BLKK), lo:hi] for j in idxs], axis=0)
                vaug_sel = jnp.concatenate(
                    [vaug_ref[hh, pl.ds(j * BLKK, BLKK), :] for j in idxs], axis=0)
                s = jax.lax.dot_general(q_i, k_sel, (((1,), (1,)), ((), ())),
                                        preferred_element_type=jnp.float32) * SCALE
                e = jnp.exp(s)
                out_aug = jax.lax.dot_general(e, vaug_sel, (((1,), (0,)), ((), ())),
                                              preferred_element_type=jnp.float32)
                kv_aug = kvb_ref[hh, pl.ds(idxs[0] * BLKK, BLKK), :]
                for j in idxs[1:]:
                    kv_aug = kv_aug + kvb_ref[hh, pl.ds(j * BLKK, BLKK), :]
                pq_i = pq_ref[pl.ds(i * BLKQ, BLKQ), lo:hi]
                out_l_aug = jax.lax.dot_general(pq_i, kvtot_augs[hh] - kv_aug,
                                                (((1,), (0,)), ((), ())),
                                                preferred_element_type=jnp.float32)
                tot = out_aug + out_l_aug
                x2_ref[g, pl.ds(i * BLKQ, BLKQ), lo:hi] = (
                    tot[:, :HD] / tot[:, HD:HD + 1])
            return 0

        jax.lax.fori_loop(0, NQ // 4, body, 0)

    @pl.when(g >= G)
    def _ln_proj():
        half = g - G
        xb = jnp.concatenate(
            [x2_ref[gg, pl.ds(half * LN_ROWS, LN_ROWS), :] for gg in range(G)],
            axis=1)
        mu = jnp.mean(xb, axis=1, keepdims=True)
        d = xb - mu
        var = jnp.mean(d * d, axis=1, keepdims=True)
        xn = d / jnp.sqrt(var + 1e-5) * g_ref[:, :] + bt_ref[:, :]
        y_ref[:, :] = jax.lax.dot_general(
            xn, wp_ref[:, :], (((1,), (1,)), ((), ())),
            preferred_element_type=jnp.float32) + bp_ref[:, :]


@functools.partial(jax.jit, static_argnames=("interpret",))
def _run(x, W_qkv, q_bias, v_bias, gamma, beta, W_proj, b_proj, interpret=False):
    x2d = x.reshape(N, C)
    qkv_bias = jnp.concatenate(
        [q_bias, jnp.zeros_like(q_bias), v_bias]).reshape(1, 3 * C)

    qkv, idx = pl.pallas_call(
        _qkv_topk_kernel,
        grid=(N_ROW_TILES,),
        in_specs=[
            pl.BlockSpec((ROW_TILE, C), lambda r: (r, 0)),
            pl.BlockSpec((3 * C, C), lambda r: (0, 0)),
            pl.BlockSpec((1, 3 * C), lambda r: (0, 0)),
        ],
        out_specs=[
            pl.BlockSpec((ROW_TILE, 3 * C), lambda r: (r, 0)),
            pl.BlockSpec((H, NQ, KSEL), lambda r: (0, 0, 0)),
        ],
        out_shape=[
            jax.ShapeDtypeStruct((N, 3 * C), jnp.float32),
            jax.ShapeDtypeStruct((H, NQ, KSEL), jnp.int32),
        ],
        scratch_shapes=[pltpu.VMEM((NQ, 2 * C), jnp.float32)],
        interpret=interpret,
    )(x2d, W_qkv, qkv_bias)

    y = pl.pallas_call(
        _attn_ln_proj_kernel,
        grid_spec=pltpu.PrefetchScalarGridSpec(
            num_scalar_prefetch=1,
            grid=(G + 2,),
            in_specs=[
                pl.BlockSpec((N, 2 * HD), lambda g, s: (0, jnp.minimum(g, G - 1))),
                pl.BlockSpec((N, 2 * HD), lambda g, s: (0, G + jnp.minimum(g, G - 1))),
                pl.BlockSpec((N, 2 * HD), lambda g, s: (0, 2 * G + jnp.minimum(g, G - 1))),
                pl.BlockSpec((1, C), lambda g, s: (0, 0)),
                pl.BlockSpec((1, C), lambda g, s: (0, 0)),
                pl.BlockSpec((C, C), lambda g, s: (0, 0)),
                pl.BlockSpec((1, C), lambda g, s: (0, 0)),
            ],
            out_specs=pl.BlockSpec((LN_ROWS, C),
                                   lambda g, s: (jnp.maximum(g - G, 0), 0)),
            scratch_shapes=[
                pltpu.VMEM((G, N, 2 * HD), jnp.float32),
                pltpu.VMEM((N, 2 * HD), jnp.float32),
                pltpu.VMEM((2, N, 2 * HD), jnp.float32),
                pltpu.VMEM((2, N, 2 * HD), jnp.float32),
            ],
        ),
        out_shape=jax.ShapeDtypeStruct((N, C), jnp.float32),
        interpret=interpret,
    )(idx, qkv, qkv, qkv, gamma.reshape(1, C), beta.reshape(1, C),
      W_proj, b_proj.reshape(1, C))

    return y.reshape(B, N, C)


def kernel(x, W_qkv, q_bias, v_bias, gamma, beta, W_proj, b_proj):
    return _run(x, W_qkv, q_bias, v_bias, gamma, beta, W_proj, b_proj)


# speedup vs baseline: 1.3188x; 1.0460x over previous
import functools
import math

import jax
import jax.numpy as jnp
from jax.experimental import pallas as pl
from jax.experimental.pallas import tpu as pltpu

B, N, C = 1, 2048, 768
H = 12
HD = C // H
BLKQ = 64
BLKK = 64
NQ = N // BLKQ
NK = N // BLKK
KSEL = max(1, int(0.25 * NK))
SCALE = 1.0 / math.sqrt(HD)

ROW_TILE = 256
N_ROW_TILES = N // ROW_TILE
POOL_PER_TILE = ROW_TILE // BLKQ
G = H // 2
LN_ROWS = N // 2


def _row_softmax(a):
    m = jnp.max(a, axis=1, keepdims=True)
    e = jnp.exp(a - m)
    return e / jnp.sum(e, axis=1, keepdims=True)


def _qkv_topk_kernel(x_ref, w_ref, b_ref, qkv_ref, idx_ref, pacc_ref):
    r = pl.program_id(0)
    t = jax.lax.dot_general(x_ref[:, :], w_ref[:, :], (((1,), (1,)), ((), ())),
                            preferred_element_type=jnp.float32) + b_ref[:, :]
    qkv_ref[:, :] = t
    qk = t[:, :2 * C]
    for b in range(POOL_PER_TILE):
        row = (jnp.sum(qk[b * BLKQ:(b + 1) * BLKQ, :], axis=0) * (1.0 / BLKQ))
        pacc_ref[pl.ds(r * POOL_PER_TILE + b, 1), :] = row[None, :]

    @pl.when(r == N_ROW_TILES - 1)
    def _():
        col = jax.lax.broadcasted_iota(jnp.int32, (NQ, NK), 1)
        for h in range(H):
            qp = pacc_ref[:, h * HD:(h + 1) * HD]
            kp = pacc_ref[:, C + h * HD:C + (h + 1) * HD]
            s = jax.lax.dot_general(qp, kp, (((1,), (1,)), ((), ())),
                                    preferred_element_type=jnp.float32)
            js = []
            for t_ in range(KSEL):
                m = jnp.max(s, axis=1, keepdims=True)
                j = jnp.min(jnp.where(s == m, col, NK), axis=1)
                js.append(j)
                s = jnp.where(col == j[:, None], -jnp.inf, s)
            idx_ref[h, :, :] = jnp.stack(js, axis=1)


def _attn_ln_proj_kernel(idx_ref, q_ref, k_ref, v_ref, g_ref, bt_ref, wp_ref,
                         bp_ref, y_ref, x2_ref, pq_ref, vaug_ref, kvb_ref):
    g = pl.program_id(0)

    @pl.when(g < G)
    def _attn():
        kvtot_augs = []
        for hh in range(2):
            lo, hi = hh * HD, (hh + 1) * HD
            k = k_ref[:, lo:hi]
            v = v_ref[:, lo:hi]
            phi = _row_softmax(k)
            pq_ref[:, lo:hi] = _row_softmax(q_ref[:, lo:hi])
            vaug_ref[hh, :, :] = jnp.concatenate(
                [v, jnp.ones((N, HD), jnp.float32)], axis=1)
            kv_tot = jax.lax.dot_general(phi, v, (((0,), (0,)), ((), ())),
                                         preferred_element_type=jnp.float32)
            s_tot = jnp.sum(phi, axis=0)
            kvtot_augs.append(jnp.concatenate(
                [kv_tot, jnp.broadcast_to(s_tot[:, None], (HD, HD))], axis=1))
            for j in range(NK):
                phi_j = phi[j * BLKK:(j + 1) * BLKK, :]
                v_j = v[j * BLKK:(j + 1) * BLKK, :]
                kv_j = jax.lax.dot_general(phi_j, v_j, (((0,), (0,)), ((), ())),
                                           preferred_element_type=jnp.float32)
                s_j = jnp.sum(phi_j, axis=0)
                kvb_ref[hh, pl.ds(j * BLKK, BLKK), :] = jnp.concatenate(
                    [kv_j, jnp.broadcast_to(s_j[:, None], (HD, HD))], axis=1)

        def body(i2, _):
            for half in range(8):
              i = i2 * 8 + half
              for hh in range(2):
                lo, hi = hh * HD, (hh + 1) * HD
                q_i = q_ref[pl.ds(i * BLKQ, BLKQ), lo:hi]
                idxs = [idx_ref[2 * g + hh, i, t] for t in range(KSEL)]
                k_sel = jnp.concatenate(
                    [k_ref[pl.ds(j * BLKK, BLKK), lo:hi] for j in idxs], axis=0)
                vaug_sel = jnp.concatenate(
                    [vaug_ref[hh, pl.ds(j * BLKK, BLKK), :] for j in idxs], axis=0)
                s = jax.lax.dot_general(q_i, k_sel, (((1,), (1,)), ((), ())),
                                        preferred_element_type=jnp.float32) * SCALE
                e = jnp.exp(s)
                out_aug = jax.lax.dot_general(e, vaug_sel, (((1,), (0,)), ((), ())),
                                              preferred_element_type=jnp.float32)
                kv_aug = kvb_ref[hh, pl.ds(idxs[0] * BLKK, BLKK), :]
                for j in idxs[1:]:
                    kv_aug = kv_aug + kvb_ref[hh, pl.ds(j * BLKK, BLKK), :]
                pq_i = pq_ref[pl.ds(i * BLKQ, BLKQ), lo:hi]
                out_l_aug = jax.lax.dot_general(pq_i, kvtot_augs[hh] - kv_aug,
                                                (((1,), (0,)), ((), ())),
                                                preferred_element_type=jnp.float32)
                tot = out_aug + out_l_aug
                x2_ref[g, pl.ds(i * BLKQ, BLKQ), lo:hi] = (
                    tot[:, :HD] / tot[:, HD:HD + 1])
            return 0

        jax.lax.fori_loop(0, NQ // 8, body, 0)

    @pl.when(g >= G)
    def _ln_proj():
        half = g - G
        xb = jnp.concatenate(
            [x2_ref[gg, pl.ds(half * LN_ROWS, LN_ROWS), :] for gg in range(G)],
            axis=1)
        mu = jnp.mean(xb, axis=1, keepdims=True)
        d = xb - mu
        var = jnp.mean(d * d, axis=1, keepdims=True)
        xn = d / jnp.sqrt(var + 1e-5) * g_ref[:, :] + bt_ref[:, :]
        y_ref[:, :] = jax.lax.dot_general(
            xn, wp_ref[:, :], (((1,), (1,)), ((), ())),
            preferred_element_type=jnp.float32) + bp_ref[:, :]


@functools.partial(jax.jit, static_argnames=("interpret",))
def _run(x, W_qkv, q_bias, v_bias, gamma, beta, W_proj, b_proj, interpret=False):
    x2d = x.reshape(N, C)
    qkv_bias = jnp.concatenate(
        [q_bias, jnp.zeros_like(q_bias), v_bias]).reshape(1, 3 * C)

    qkv, idx = pl.pallas_call(
        _qkv_topk_kernel,
        grid=(N_ROW_TILES,),
        in_specs=[
            pl.BlockSpec((ROW_TILE, C), lambda r: (r, 0)),
            pl.BlockSpec((3 * C, C), lambda r: (0, 0)),
            pl.BlockSpec((1, 3 * C), lambda r: (0, 0)),
        ],
        out_specs=[
            pl.BlockSpec((ROW_TILE, 3 * C), lambda r: (r, 0)),
            pl.BlockSpec((H, NQ, KSEL), lambda r: (0, 0, 0)),
        ],
        out_shape=[
            jax.ShapeDtypeStruct((N, 3 * C), jnp.float32),
            jax.ShapeDtypeStruct((H, NQ, KSEL), jnp.int32),
        ],
        scratch_shapes=[pltpu.VMEM((NQ, 2 * C), jnp.float32)],
        interpret=interpret,
    )(x2d, W_qkv, qkv_bias)

    y = pl.pallas_call(
        _attn_ln_proj_kernel,
        grid_spec=pltpu.PrefetchScalarGridSpec(
            num_scalar_prefetch=1,
            grid=(G + 2,),
            in_specs=[
                pl.BlockSpec((N, 2 * HD), lambda g, s: (0, jnp.minimum(g, G - 1))),
                pl.BlockSpec((N, 2 * HD), lambda g, s: (0, G + jnp.minimum(g, G - 1))),
                pl.BlockSpec((N, 2 * HD), lambda g, s: (0, 2 * G + jnp.minimum(g, G - 1))),
                pl.BlockSpec((1, C), lambda g, s: (0, 0)),
                pl.BlockSpec((1, C), lambda g, s: (0, 0)),
                pl.BlockSpec((C, C), lambda g, s: (0, 0)),
                pl.BlockSpec((1, C), lambda g, s: (0, 0)),
            ],
            out_specs=pl.BlockSpec((LN_ROWS, C),
                                   lambda g, s: (jnp.maximum(g - G, 0), 0)),
            scratch_shapes=[
                pltpu.VMEM((G, N, 2 * HD), jnp.float32),
                pltpu.VMEM((N, 2 * HD), jnp.float32),
                pltpu.VMEM((2, N, 2 * HD), jnp.float32),
                pltpu.VMEM((2, N, 2 * HD), jnp.float32),
            ],
        ),
        out_shape=jax.ShapeDtypeStruct((N, C), jnp.float32),
        interpret=interpret,
    )(idx, qkv, qkv, qkv, gamma.reshape(1, C), beta.reshape(1, C),
      W_proj, b_proj.reshape(1, C))

    return y.reshape(B, N, C)


def kernel(x, W_qkv, q_bias, v_bias, gamma, beta, W_proj, b_proj):
    return _run(x, W_qkv, q_bias, v_bias, gamma, beta, W_proj, b_proj)


# speedup vs baseline: 1.3459x; 1.0206x over previous
import functools
import math

import jax
import jax.numpy as jnp
from jax.experimental import pallas as pl
from jax.experimental.pallas import tpu as pltpu

B, N, C = 1, 2048, 768
H = 12
HD = C // H
BLKQ = 64
BLKK = 64
NQ = N // BLKQ
NK = N // BLKK
KSEL = max(1, int(0.25 * NK))
SCALE = 1.0 / math.sqrt(HD)

ROW_TILE = 256
N_ROW_TILES = N // ROW_TILE
POOL_PER_TILE = ROW_TILE // BLKQ
G = H // 2
LN_ROWS = N // 2


def _row_softmax(a):
    m = jnp.max(a, axis=1, keepdims=True)
    e = jnp.exp(a - m)
    return e / jnp.sum(e, axis=1, keepdims=True)


def _qkv_topk_kernel(x_ref, w_ref, b_ref, qkv_ref, idx_ref, pacc_ref):
    r = pl.program_id(0)
    t = jax.lax.dot_general(x_ref[:, :], w_ref[:, :], (((1,), (1,)), ((), ())),
                            preferred_element_type=jnp.float32) + b_ref[:, :]
    qkv_ref[:, :] = t
    qk = t[:, :2 * C]
    for b in range(POOL_PER_TILE):
        row = (jnp.sum(qk[b * BLKQ:(b + 1) * BLKQ, :], axis=0) * (1.0 / BLKQ))
        pacc_ref[pl.ds(r * POOL_PER_TILE + b, 1), :] = row[None, :]

    @pl.when(r == N_ROW_TILES - 1)
    def _():
        col = jax.lax.broadcasted_iota(jnp.int32, (NQ, NK), 1)
        for h in range(H):
            qp = pacc_ref[:, h * HD:(h + 1) * HD]
            kp = pacc_ref[:, C + h * HD:C + (h + 1) * HD]
            s = jax.lax.dot_general(qp, kp, (((1,), (1,)), ((), ())),
                                    preferred_element_type=jnp.float32)
            js = []
            for t_ in range(KSEL):
                m = jnp.max(s, axis=1, keepdims=True)
                j = jnp.min(jnp.where(s == m, col, NK), axis=1)
                js.append(j)
                s = jnp.where(col == j[:, None], -jnp.inf, s)
            idx_ref[h, :, :] = jnp.stack(js, axis=1)


def _attn_ln_proj_kernel(idx_ref, q_ref, k_ref, v_ref, g_ref, bt_ref, wp_ref,
                         bp_ref, y_ref, x2_ref, pq_ref, vaug_ref, kvb_ref):
    g = pl.program_id(0)

    @pl.when(g < G)
    def _attn():
        kvtot_augs = []
        for hh in range(2):
            lo, hi = hh * HD, (hh + 1) * HD
            k = k_ref[:, lo:hi]
            v = v_ref[:, lo:hi]
            phi = _row_softmax(k)
            pq_ref[:, lo:hi] = _row_softmax(q_ref[:, lo:hi])
            vaug_ref[hh, :, :] = jnp.concatenate(
                [v, jnp.ones((N, HD), jnp.float32)], axis=1)
            kv_tot = jax.lax.dot_general(phi, v, (((0,), (0,)), ((), ())),
                                         preferred_element_type=jnp.float32)
            s_tot = jnp.sum(phi, axis=0)
            kvtot_augs.append(jnp.concatenate(
                [kv_tot, jnp.broadcast_to(s_tot[:, None], (HD, HD))], axis=1))
            for j in range(NK):
                phi_j = phi[j * BLKK:(j + 1) * BLKK, :]
                v_j = v[j * BLKK:(j + 1) * BLKK, :]
                kv_j = jax.lax.dot_general(phi_j, v_j, (((0,), (0,)), ((), ())),
                                           preferred_element_type=jnp.float32)
                s_j = jnp.sum(phi_j, axis=0)
                kvb_ref[hh, pl.ds(j * BLKK, BLKK), :] = jnp.concatenate(
                    [kv_j, jnp.broadcast_to(s_j[:, None], (HD, HD))], axis=1)

        def body(i2, _):
            for half in range(16):
              i = i2 * 16 + half
              for hh in range(2):
                lo, hi = hh * HD, (hh + 1) * HD
                q_i = q_ref[pl.ds(i * BLKQ, BLKQ), lo:hi]
                idxs = [idx_ref[2 * g + hh, i, t] for t in range(KSEL)]
                k_sel = jnp.concatenate(
                    [k_ref[pl.ds(j * BLKK, BLKK), lo:hi] for j in idxs], axis=0)
                vaug_sel = jnp.concatenate(
                    [vaug_ref[hh, pl.ds(j * BLKK, BLKK), :] for j in idxs], axis=0)
                s = jax.lax.dot_general(q_i, k_sel, (((1,), (1,)), ((), ())),
                                        preferred_element_type=jnp.float32) * SCALE
                e = jnp.exp(s)
                out_aug = jax.lax.dot_general(e, vaug_sel, (((1,), (0,)), ((), ())),
                                              preferred_element_type=jnp.float32)
                kv_aug = kvb_ref[hh, pl.ds(idxs[0] * BLKK, BLKK), :]
                for j in idxs[1:]:
                    kv_aug = kv_aug + kvb_ref[hh, pl.ds(j * BLKK, BLKK), :]
                pq_i = pq_ref[pl.ds(i * BLKQ, BLKQ), lo:hi]
                out_l_aug = jax.lax.dot_general(pq_i, kvtot_augs[hh] - kv_aug,
                                                (((1,), (0,)), ((), ())),
                                                preferred_element_type=jnp.float32)
                tot = out_aug + out_l_aug
                x2_ref[g, pl.ds(i * BLKQ, BLKQ), lo:hi] = (
                    tot[:, :HD] / tot[:, HD:HD + 1])
            return 0

        jax.lax.fori_loop(0, NQ // 16, body, 0)

    @pl.when(g >= G)
    def _ln_proj():
        half = g - G
        xb = jnp.concatenate(
            [x2_ref[gg, pl.ds(half * LN_ROWS, LN_ROWS), :] for gg in range(G)],
            axis=1)
        mu = jnp.mean(xb, axis=1, keepdims=True)
        d = xb - mu
        var = jnp.mean(d * d, axis=1, keepdims=True)
        xn = d / jnp.sqrt(var + 1e-5) * g_ref[:, :] + bt_ref[:, :]
        y_ref[:, :] = jax.lax.dot_general(
            xn, wp_ref[:, :], (((1,), (1,)), ((), ())),
            preferred_element_type=jnp.float32) + bp_ref[:, :]


@functools.partial(jax.jit, static_argnames=("interpret",))
def _run(x, W_qkv, q_bias, v_bias, gamma, beta, W_proj, b_proj, interpret=False):
    x2d = x.reshape(N, C)
    qkv_bias = jnp.concatenate(
        [q_bias, jnp.zeros_like(q_bias), v_bias]).reshape(1, 3 * C)

    qkv, idx = pl.pallas_call(
        _qkv_topk_kernel,
        grid=(N_ROW_TILES,),
        in_specs=[
            pl.BlockSpec((ROW_TILE, C), lambda r: (r, 0)),
            pl.BlockSpec((3 * C, C), lambda r: (0, 0)),
            pl.BlockSpec((1, 3 * C), lambda r: (0, 0)),
        ],
        out_specs=[
            pl.BlockSpec((ROW_TILE, 3 * C), lambda r: (r, 0)),
            pl.BlockSpec((H, NQ, KSEL), lambda r: (0, 0, 0)),
        ],
        out_shape=[
            jax.ShapeDtypeStruct((N, 3 * C), jnp.float32),
            jax.ShapeDtypeStruct((H, NQ, KSEL), jnp.int32),
        ],
        scratch_shapes=[pltpu.VMEM((NQ, 2 * C), jnp.float32)],
        interpret=interpret,
    )(x2d, W_qkv, qkv_bias)

    y = pl.pallas_call(
        _attn_ln_proj_kernel,
        grid_spec=pltpu.PrefetchScalarGridSpec(
            num_scalar_prefetch=1,
            grid=(G + 2,),
            in_specs=[
                pl.BlockSpec((N, 2 * HD), lambda g, s: (0, jnp.minimum(g, G - 1))),
                pl.BlockSpec((N, 2 * HD), lambda g, s: (0, G + jnp.minimum(g, G - 1))),
                pl.BlockSpec((N, 2 * HD), lambda g, s: (0, 2 * G + jnp.minimum(g, G - 1))),
                pl.BlockSpec((1, C), lambda g, s: (0, 0)),
                pl.BlockSpec((1, C), lambda g, s: (0, 0)),
                pl.BlockSpec((C, C), lambda g, s: (0, 0)),
                pl.BlockSpec((1, C), lambda g, s: (0, 0)),
            ],
            out_specs=pl.BlockSpec((LN_ROWS, C),
                                   lambda g, s: (jnp.maximum(g - G, 0), 0)),
            scratch_shapes=[
                pltpu.VMEM((G, N, 2 * HD), jnp.float32),
                pltpu.VMEM((N, 2 * HD), jnp.float32),
                pltpu.VMEM((2, N, 2 * HD), jnp.float32),
                pltpu.VMEM((2, N, 2 * HD), jnp.float32),
            ],
        ),
        out_shape=jax.ShapeDtypeStruct((N, C), jnp.float32),
        interpret=interpret,
    )(idx, qkv, qkv, qkv, gamma.reshape(1, C), beta.reshape(1, C),
      W_proj, b_proj.reshape(1, C))

    return y.reshape(B, N, C)


def kernel(x, W_qkv, q_bias, v_bias, gamma, beta, W_proj, b_proj):
    return _run(x, W_qkv, q_bias, v_bias, gamma, beta, W_proj, b_proj)


# speedup vs baseline: 1.3649x; 1.0141x over previous
import functools
import math

import jax
import jax.numpy as jnp
from jax.experimental import pallas as pl
from jax.experimental.pallas import tpu as pltpu

B, N, C = 1, 2048, 768
H = 12
HD = C // H
BLKQ = 64
BLKK = 64
NQ = N // BLKQ
NK = N // BLKK
KSEL = max(1, int(0.25 * NK))
SCALE = 1.0 / math.sqrt(HD)

ROW_TILE = 512
N_ROW_TILES = N // ROW_TILE
POOL_PER_TILE = ROW_TILE // BLKQ
G = H // 2
LN_ROWS = N // 2


def _row_softmax(a):
    m = jnp.max(a, axis=1, keepdims=True)
    e = jnp.exp(a - m)
    return e / jnp.sum(e, axis=1, keepdims=True)


def _qkv_topk_kernel(x_ref, w_ref, b_ref, qkv_ref, idx_ref, pacc_ref):
    r = pl.program_id(0)
    t = jax.lax.dot_general(x_ref[:, :], w_ref[:, :], (((1,), (1,)), ((), ())),
                            preferred_element_type=jnp.float32) + b_ref[:, :]
    qkv_ref[:, :] = t
    qk = t[:, :2 * C]
    for b in range(POOL_PER_TILE):
        row = (jnp.sum(qk[b * BLKQ:(b + 1) * BLKQ, :], axis=0) * (1.0 / BLKQ))
        pacc_ref[pl.ds(r * POOL_PER_TILE + b, 1), :] = row[None, :]

    @pl.when(r == N_ROW_TILES - 1)
    def _():
        col = jax.lax.broadcasted_iota(jnp.int32, (NQ, NK), 1)
        for h in range(H):
            qp = pacc_ref[:, h * HD:(h + 1) * HD]
            kp = pacc_ref[:, C + h * HD:C + (h + 1) * HD]
            s = jax.lax.dot_general(qp, kp, (((1,), (1,)), ((), ())),
                                    preferred_element_type=jnp.float32)
            js = []
            for t_ in range(KSEL):
                m = jnp.max(s, axis=1, keepdims=True)
                j = jnp.min(jnp.where(s == m, col, NK), axis=1)
                js.append(j)
                s = jnp.where(col == j[:, None], -jnp.inf, s)
            idx_ref[h, :, :] = jnp.stack(js, axis=1)


def _attn_ln_proj_kernel(idx_ref, q_ref, k_ref, v_ref, g_ref, bt_ref, wp_ref,
                         bp_ref, y_ref, x2_ref, pq_ref, vaug_ref, kvb_ref):
    g = pl.program_id(0)

    @pl.when(g < G)
    def _attn():
        kvtot_augs = []
        for hh in range(2):
            lo, hi = hh * HD, (hh + 1) * HD
            k = k_ref[:, lo:hi]
            v = v_ref[:, lo:hi]
            phi = _row_softmax(k)
            pq_ref[:, lo:hi] = _row_softmax(q_ref[:, lo:hi])
            vaug_ref[hh, :, :] = jnp.concatenate(
                [v, jnp.ones((N, HD), jnp.float32)], axis=1)
            kv_tot = jax.lax.dot_general(phi, v, (((0,), (0,)), ((), ())),
                                         preferred_element_type=jnp.float32)
            s_tot = jnp.sum(phi, axis=0)
            kvtot_augs.append(jnp.concatenate(
                [kv_tot, jnp.broadcast_to(s_tot[:, None], (HD, HD))], axis=1))
            for j in range(NK):
                phi_j = phi[j * BLKK:(j + 1) * BLKK, :]
                v_j = v[j * BLKK:(j + 1) * BLKK, :]
                kv_j = jax.lax.dot_general(phi_j, v_j, (((0,), (0,)), ((), ())),
                                           preferred_element_type=jnp.float32)
                s_j = jnp.sum(phi_j, axis=0)
                kvb_ref[hh, pl.ds(j * BLKK, BLKK), :] = jnp.concatenate(
                    [kv_j, jnp.broadcast_to(s_j[:, None], (HD, HD))], axis=1)

        def body(i2, _):
            for half in range(16):
              i = i2 * 16 + half
              for hh in range(2):
                lo, hi = hh * HD, (hh + 1) * HD
                q_i = q_ref[pl.ds(i * BLKQ, BLKQ), lo:hi]
                idxs = [idx_ref[2 * g + hh, i, t] for t in range(KSEL)]
                k_sel = jnp.concatenate(
                    [k_ref[pl.ds(j * BLKK, BLKK), lo:hi] for j in idxs], axis=0)
                vaug_sel = jnp.concatenate(
                    [vaug_ref[hh, pl.ds(j * BLKK, BLKK), :] for j in idxs], axis=0)
                s = jax.lax.dot_general(q_i, k_sel, (((1,), (1,)), ((), ())),
                                        preferred_element_type=jnp.float32) * SCALE
                e = jnp.exp(s)
                out_aug = jax.lax.dot_general(e, vaug_sel, (((1,), (0,)), ((), ())),
                                              preferred_element_type=jnp.float32)
                kv_aug = kvb_ref[hh, pl.ds(idxs[0] * BLKK, BLKK), :]
                for j in idxs[1:]:
                    kv_aug = kv_aug + kvb_ref[hh, pl.ds(j * BLKK, BLKK), :]
                pq_i = pq_ref[pl.ds(i * BLKQ, BLKQ), lo:hi]
                out_l_aug = jax.lax.dot_general(pq_i, kvtot_augs[hh] - kv_aug,
                                                (((1,), (0,)), ((), ())),
                                                preferred_element_type=jnp.float32)
                tot = out_aug + out_l_aug
                x2_ref[g, pl.ds(i * BLKQ, BLKQ), lo:hi] = (
                    tot[:, :HD] / tot[:, HD:HD + 1])
            return 0

        jax.lax.fori_loop(0, NQ // 16, body, 0)

    @pl.when(g >= G)
    def _ln_proj():
        half = g - G
        xb = jnp.concatenate(
            [x2_ref[gg, pl.ds(half * LN_ROWS, LN_ROWS), :] for gg in range(G)],
            axis=1)
        mu = jnp.mean(xb, axis=1, keepdims=True)
        d = xb - mu
        var = jnp.mean(d * d, axis=1, keepdims=True)
        xn = d / jnp.sqrt(var + 1e-5) * g_ref[:, :] + bt_ref[:, :]
        y_ref[:, :] = jax.lax.dot_general(
            xn, wp_ref[:, :], (((1,), (1,)), ((), ())),
            preferred_element_type=jnp.float32) + bp_ref[:, :]


@functools.partial(jax.jit, static_argnames=("interpret",))
def _run(x, W_qkv, q_bias, v_bias, gamma, beta, W_proj, b_proj, interpret=False):
    x2d = x.reshape(N, C)
    qkv_bias = jnp.concatenate(
        [q_bias, jnp.zeros_like(q_bias), v_bias]).reshape(1, 3 * C)

    qkv, idx = pl.pallas_call(
        _qkv_topk_kernel,
        grid=(N_ROW_TILES,),
        in_specs=[
            pl.BlockSpec((ROW_TILE, C), lambda r: (r, 0)),
            pl.BlockSpec((3 * C, C), lambda r: (0, 0)),
            pl.BlockSpec((1, 3 * C), lambda r: (0, 0)),
        ],
        out_specs=[
            pl.BlockSpec((ROW_TILE, 3 * C), lambda r: (r, 0)),
            pl.BlockSpec((H, NQ, KSEL), lambda r: (0, 0, 0)),
        ],
        out_shape=[
            jax.ShapeDtypeStruct((N, 3 * C), jnp.float32),
            jax.ShapeDtypeStruct((H, NQ, KSEL), jnp.int32),
        ],
        scratch_shapes=[pltpu.VMEM((NQ, 2 * C), jnp.float32)],
        interpret=interpret,
    )(x2d, W_qkv, qkv_bias)

    y = pl.pallas_call(
        _attn_ln_proj_kernel,
        grid_spec=pltpu.PrefetchScalarGridSpec(
            num_scalar_prefetch=1,
            grid=(G + 2,),
            in_specs=[
                pl.BlockSpec((N, 2 * HD), lambda g, s: (0, jnp.minimum(g, G - 1))),
                pl.BlockSpec((N, 2 * HD), lambda g, s: (0, G + jnp.minimum(g, G - 1))),
                pl.BlockSpec((N, 2 * HD), lambda g, s: (0, 2 * G + jnp.minimum(g, G - 1))),
                pl.BlockSpec((1, C), lambda g, s: (0, 0)),
                pl.BlockSpec((1, C), lambda g, s: (0, 0)),
                pl.BlockSpec((C, C), lambda g, s: (0, 0)),
                pl.BlockSpec((1, C), lambda g, s: (0, 0)),
            ],
            out_specs=pl.BlockSpec((LN_ROWS, C),
                                   lambda g, s: (jnp.maximum(g - G, 0), 0)),
            scratch_shapes=[
                pltpu.VMEM((G, N, 2 * HD), jnp.float32),
                pltpu.VMEM((N, 2 * HD), jnp.float32),
                pltpu.VMEM((2, N, 2 * HD), jnp.float32),
                pltpu.VMEM((2, N, 2 * HD), jnp.float32),
            ],
        ),
        out_shape=jax.ShapeDtypeStruct((N, C), jnp.float32),
        interpret=interpret,
    )(idx, qkv, qkv, qkv, gamma.reshape(1, C), beta.reshape(1, C),
      W_proj, b_proj.reshape(1, C))

    return y.reshape(B, N, C)


def kernel(x, W_qkv, q_bias, v_bias, gamma, beta, W_proj, b_proj):
    return _run(x, W_qkv, q_bias, v_bias, gamma, beta, W_proj, b_proj)
